# baseline (speedup 1.0000x reference)
import functools

import jax
import jax.numpy as jnp
from jax import lax
from jax.experimental import pallas as pl
from jax.experimental.pallas import tpu as pltpu
from jax.experimental.pallas import tpu_sc as plsc

N = 10000
E = 320000
D_IN = 128
D_HID = 128
D_OUT = 64

NC = 2
NS = 16
NW = NC * NS
CH = 128
NCH = 84
E_PAD = NW * NCH * CH
N_ACC = 10016
ROWS_PER_TILE_ACC = N_ACC // NS
N_DEG = 10112
ROWS_PER_TILE_DEG = N_DEG // NS

_mesh = plsc.VectorSubcoreMesh(core_axis_name="c", subcore_axis_name="s")


_NBUF = 3
_G = 6
_NGRP = NCH // _G


def _make_edge_kernel(d):
    @functools.partial(
        pl.kernel,
        out_type=jax.ShapeDtypeStruct((NC, N_ACC, d), jnp.float32),
        mesh=_mesh,
        compiler_params=pltpu.CompilerParams(use_tc_tiling_on_sc=False),
        scratch_types=[
            pltpu.VMEM((_G, CH), jnp.int32),
            pltpu.VMEM((_G, CH), jnp.int32),
            [pltpu.VMEM((CH, d), jnp.float32)] * _NBUF,
            pltpu.VMEM_SHARED((N_ACC, d), jnp.float32),
            [pltpu.SemaphoreType.DMA] * _NBUF,
            [pltpu.SemaphoreType.DMA] * _NBUF,
        ],
    )
    def edge_kernel(tab_hbm, src_hbm, dst_hbm, zeros_hbm, out_hbm,
                    src_v, dst_v, bufs, acc, gsems, ssems):
        c = lax.axis_index("c")
        s = lax.axis_index("s")
        w = c * NS + s
        base = s * ROWS_PER_TILE_ACC
        pltpu.sync_copy(zeros_hbm, acc.at[pl.ds(base, ROWS_PER_TILE_ACC)])
        plsc.subcore_barrier()

        def group(g, _):
            pltpu.sync_copy(src_hbm.at[pl.ds(w * NCH + g * _G, _G)], src_v)
            pltpu.sync_copy(dst_hbm.at[pl.ds(w * NCH + g * _G, _G)], dst_v)
            for t in range(_G // _NBUF):
                gds = []
                for u in range(_NBUF):
                    def drain(u=u):
                        pltpu.make_async_copy(tab_hbm.at[pl.ds(0, CH)],
                                              bufs[u], ssems[u]).wait()
                    if t == 0:
                        pl.when(g > 0)(drain)
                    else:
                        drain()
                    j = t * _NBUF + u
                    gds.append(pltpu.async_copy(tab_hbm.at[src_v.at[j]],
                                                bufs[u], gsems[u]))
                for u in range(_NBUF):
                    j = t * _NBUF + u
                    gds[u].wait()
                    pltpu.async_copy(bufs[u], acc.at[dst_v.at[j]], ssems[u],
                                     add=True)
            return 0

        lax.fori_loop(0, _NGRP, group, 0)
        for u in range(_NBUF):
            pltpu.make_async_copy(tab_hbm.at[pl.ds(0, CH)], bufs[u],
                                  ssems[u]).wait()
        plsc.subcore_barrier()
        pltpu.sync_copy(acc.at[pl.ds(base, ROWS_PER_TILE_ACC)],
                        out_hbm.at[c, pl.ds(base, ROWS_PER_TILE_ACC)])

    return edge_kernel


_edge_kernel_128 = _make_edge_kernel(D_HID)
_edge_kernel_64 = _make_edge_kernel(D_OUT)


_DEGW = 32

@functools.partial(
    pl.kernel,
    out_type=jax.ShapeDtypeStruct((NC, N_DEG, _DEGW), jnp.float32),
    mesh=_mesh,
    compiler_params=pltpu.CompilerParams(use_tc_tiling_on_sc=False),
    scratch_types=[
        [pltpu.VMEM((_G, CH), jnp.int32)] * 2,
        pltpu.VMEM((CH, _DEGW), jnp.float32),
        pltpu.VMEM_SHARED((N_DEG, _DEGW), jnp.float32),
        pltpu.SemaphoreType.DMA,
    ],
)
def _deg_kernel(dst_hbm, ones_hbm, zeros_hbm, out_hbm, dst_v, buf, acc, sem):
    c = lax.axis_index("c")
    s = lax.axis_index("s")
    w = c * NS + s
    base = s * ROWS_PER_TILE_DEG
    pltpu.sync_copy(zeros_hbm, acc.at[pl.ds(base, ROWS_PER_TILE_DEG)])
    pltpu.sync_copy(ones_hbm, buf)
    plsc.subcore_barrier()

    def group(g, _):
        for r in range(2):
            @pl.when(jnp.logical_and(g >= 2, g % 2 == r))
            def _():
                for _k in range(_G):
                    pltpu.make_async_copy(ones_hbm, buf, sem).wait()
                pltpu.sync_copy(dst_hbm.at[pl.ds(w * NCH + g * _G, _G)],
                                dst_v[r])

        @pl.when(g < 2)
        def _():
            for r in range(2):
                @pl.when(g == r)
                def _():
                    pltpu.sync_copy(dst_hbm.at[pl.ds(w * NCH + g * _G, _G)],
                                    dst_v[r])

        for r in range(2):
            @pl.when(g % 2 == r)
            def _():
                for j in range(_G):
                    pltpu.async_copy(buf, acc.at[dst_v[r].at[j]], sem,
                                     add=True)
        return 0

    lax.fori_loop(0, _NGRP, group, 0)
    for _k in range(2 * _G):
        pltpu.make_async_copy(ones_hbm, buf, sem).wait()
    plsc.subcore_barrier()
    pltpu.sync_copy(acc.at[pl.ds(base, ROWS_PER_TILE_DEG)],
                    out_hbm.at[c, pl.ds(base, ROWS_PER_TILE_DEG)])


_BLK = 400
_GRID = N // _BLK


def _tc1_body(deg_ref, x_ref, w_ref, dis_ref, hp_ref):
    deg = deg_ref[0] + deg_ref[1] + 1.0
    dis = lax.rsqrt(deg)
    dis_ref[...] = dis
    h = jnp.dot(x_ref[...], w_ref[...], preferred_element_type=jnp.float32)
    hp_ref[...] = h * dis[:, 0:1]


def _tc_mid_body(acc_ref, hp_ref, dis_ref, b_ref, w_ref, out_ref):
    dis = dis_ref[...][:, 0:1]
    t = (acc_ref[0] + acc_ref[1] + hp_ref[...]) * dis + b_ref[...]
    t = jnp.maximum(t, 0.0)
    out_ref[...] = jnp.dot(t, w_ref[...], preferred_element_type=jnp.float32) * dis


def _tc_final_body(acc_ref, hp_ref, dis_ref, b_ref, out_ref):
    dis = dis_ref[...][:, 0:1]
    out_ref[...] = (acc_ref[0] + acc_ref[1] + hp_ref[...]) * dis + b_ref[...]


def _tc1(deg, x, w):
    return pl.pallas_call(
        _tc1_body,
        grid=(_GRID,),
        in_specs=[
            pl.BlockSpec((NC, _BLK, 16), lambda i: (0, i, 0)),
            pl.BlockSpec((_BLK, D_IN), lambda i: (i, 0)),
            pl.BlockSpec((D_IN, D_HID), lambda i: (0, 0)),
        ],
        out_specs=[
            pl.BlockSpec((_BLK, 16), lambda i: (i, 0)),
            pl.BlockSpec((_BLK, D_HID), lambda i: (i, 0)),
        ],
        out_shape=[
            jax.ShapeDtypeStruct((N, 16), jnp.float32),
            jax.ShapeDtypeStruct((N, D_HID), jnp.float32),
        ],
    )(deg, x, w)


def _tc_mid(acc, hp, dis, b, w):
    d_in, d_out = w.shape
    return pl.pallas_call(
        _tc_mid_body,
        grid=(_GRID,),
        in_specs=[
            pl.BlockSpec((NC, _BLK, d_in), lambda i: (0, i, 0)),
            pl.BlockSpec((_BLK, d_in), lambda i: (i, 0)),
            pl.BlockSpec((_BLK, 16), lambda i: (i, 0)),
            pl.BlockSpec((1, d_in), lambda i: (0, 0)),
            pl.BlockSpec((d_in, d_out), lambda i: (0, 0)),
        ],
        out_specs=pl.BlockSpec((_BLK, d_out), lambda i: (i, 0)),
        out_shape=jax.ShapeDtypeStruct((N, d_out), jnp.float32),
    )(acc, hp, dis, b, w)


def _tc_final(acc, hp, dis, b):
    d = hp.shape[1]
    return pl.pallas_call(
        _tc_final_body,
        grid=(_GRID,),
        in_specs=[
            pl.BlockSpec((NC, _BLK, d), lambda i: (0, i, 0)),
            pl.BlockSpec((_BLK, d), lambda i: (i, 0)),
            pl.BlockSpec((_BLK, 16), lambda i: (i, 0)),
            pl.BlockSpec((1, d), lambda i: (0, 0)),
        ],
        out_specs=pl.BlockSpec((_BLK, d), lambda i: (i, 0)),
        out_shape=jax.ShapeDtypeStruct((N, d), jnp.float32),
    )(acc, hp, dis, b)


def kernel(x, edge_index, W1, b1, W2, b2, W_out, b_out):
    pad = E_PAD - E
    pad_i = jnp.arange(pad, dtype=jnp.int32)
    src = jnp.concatenate([edge_index[0], jnp.full((pad,), N, jnp.int32)])
    dst = jnp.concatenate([edge_index[1], pad_i % N])
    dstd = jnp.concatenate([edge_index[1], N + pad_i % (N_DEG - N)])
    src = src.reshape(NW * NCH, CH)
    dst = dst.reshape(NW * NCH, CH)
    dstd = dstd.reshape(NW * NCH, CH)
    zeros128 = jnp.zeros((ROWS_PER_TILE_DEG, D_HID), jnp.float32)
    ones_deg = jnp.ones((CH, _DEGW), jnp.float32)
    pad_rows = ((0, N_ACC - N), (0, 0))

    deg = _deg_kernel(dstd, ones_deg, zeros128[:, :_DEGW])[:, :N, :16]
    dis, h1p = _tc1(deg, x, W1)
    h1z = jnp.pad(h1p, pad_rows)
    acc1 = _edge_kernel_128(h1z, src, dst, zeros128[:ROWS_PER_TILE_ACC])[:, :N]
    h2p = _tc_mid(acc1, h1p, dis, b1.reshape(1, D_HID), W2)
    h2z = jnp.pad(h2p, pad_rows)
    acc2 = _edge_kernel_128(h2z, src, dst, zeros128[:ROWS_PER_TILE_ACC])[:, :N]
    h3p = _tc_mid(acc2, h2p, dis, b2.reshape(1, D_HID), W_out)
    h3z = jnp.pad(h3p, pad_rows)
    acc3 = _edge_kernel_64(h3z, src, dst, zeros128[:ROWS_PER_TILE_ACC, :D_OUT])[:, :N]
    out = _tc_final(acc3, h3p, dis, b_out.reshape(1, D_OUT))
    return out


# speedup vs baseline: 5.3638x; 5.3638x over previous
import functools

import jax
import jax.numpy as jnp
from jax import lax
from jax.experimental import pallas as pl
from jax.experimental.pallas import tpu as pltpu
from jax.experimental.pallas import tpu_sc as plsc

N = 10000
E = 320000
D_IN = 128
D_HID = 128
D_OUT = 64

NC = 2
NS = 16
NW = NC * NS
CH = 128
NCH = 80
E_PAD = NW * NCH * CH
N_ACC = 10112
ROWS_PER_TILE_ACC = N_ACC // NS

_mesh = plsc.VectorSubcoreMesh(core_axis_name="c", subcore_axis_name="s")


_NBUF = 2
_G = 8
_NGRP = NCH // _G


def _make_edge_kernel(d):
    @functools.partial(
        pl.kernel,
        out_type=jax.ShapeDtypeStruct((NC, N_ACC, d), jnp.float32),
        mesh=_mesh,
        compiler_params=pltpu.CompilerParams(use_tc_tiling_on_sc=False),
        scratch_types=[
            pltpu.VMEM((_G, CH), jnp.int32),
            pltpu.VMEM((_G, CH), jnp.int32),
            [pltpu.VMEM((CH, d), jnp.float32)] * _NBUF,
            pltpu.VMEM_SHARED((N_ACC, d), jnp.float32),
            [pltpu.SemaphoreType.DMA] * _NBUF,
            [pltpu.SemaphoreType.DMA] * _NBUF,
        ],
    )
    def edge_kernel(tab_hbm, src_hbm, dst_hbm, zeros_hbm, out_hbm,
                    src_v, dst_v, bufs, acc, gsems, ssems):
        c = lax.axis_index("c")
        s = lax.axis_index("s")
        w = c * NS + s
        base = s * ROWS_PER_TILE_ACC
        pltpu.sync_copy(zeros_hbm, acc.at[pl.ds(base, ROWS_PER_TILE_ACC)])
        plsc.subcore_barrier()

        def group(g, _):
            pltpu.sync_copy(src_hbm.at[pl.ds(w * NCH + g * _G, _G)], src_v)
            pltpu.sync_copy(dst_hbm.at[pl.ds(w * NCH + g * _G, _G)], dst_v)
            for t in range(_G // _NBUF):
                gds = []
                for u in range(_NBUF):
                    def drain(u=u):
                        pltpu.make_async_copy(tab_hbm.at[pl.ds(0, CH)],
                                              bufs[u], ssems[u]).wait()
                    if t == 0:
                        pl.when(g > 0)(drain)
                    else:
                        drain()
                    j = t * _NBUF + u
                    gds.append(pltpu.async_copy(tab_hbm.at[src_v.at[j]],
                                                bufs[u], gsems[u]))
                for u in range(_NBUF):
                    j = t * _NBUF + u
                    gds[u].wait()
                    pltpu.async_copy(bufs[u], acc.at[dst_v.at[j]], ssems[u],
                                     add=True)
            return 0

        lax.fori_loop(0, _NGRP, group, 0)
        for u in range(_NBUF):
            pltpu.make_async_copy(tab_hbm.at[pl.ds(0, CH)], bufs[u],
                                  ssems[u]).wait()
        plsc.subcore_barrier()
        pltpu.sync_copy(acc.at[pl.ds(base, ROWS_PER_TILE_ACC)],
                        out_hbm.at[c, pl.ds(base, ROWS_PER_TILE_ACC)])

    return edge_kernel


_edge_kernel_128 = _make_edge_kernel(D_HID)
_edge_kernel_64 = _make_edge_kernel(D_OUT)


_DEGW = 32

@functools.partial(
    pl.kernel,
    out_type=jax.ShapeDtypeStruct((NC, N_ACC, _DEGW), jnp.float32),
    mesh=_mesh,
    compiler_params=pltpu.CompilerParams(use_tc_tiling_on_sc=False),
    scratch_types=[
        [pltpu.VMEM((_G, CH), jnp.int32)] * 2,
        pltpu.VMEM((CH, _DEGW), jnp.float32),
        pltpu.VMEM_SHARED((N_ACC, _DEGW), jnp.float32),
        pltpu.SemaphoreType.DMA,
    ],
)
def _deg_kernel(dst_hbm, ones_hbm, zeros_hbm, out_hbm, dst_v, buf, acc, sem):
    c = lax.axis_index("c")
    s = lax.axis_index("s")
    w = c * NS + s
    base = s * ROWS_PER_TILE_ACC
    pltpu.sync_copy(zeros_hbm, acc.at[pl.ds(base, ROWS_PER_TILE_ACC)])
    pltpu.sync_copy(ones_hbm, buf)
    plsc.subcore_barrier()

    def group(g, _):
        for r in range(2):
            @pl.when(jnp.logical_and(g >= 2, g % 2 == r))
            def _():
                for _k in range(_G):
                    pltpu.make_async_copy(ones_hbm, buf, sem).wait()
                pltpu.sync_copy(dst_hbm.at[pl.ds(w * NCH + g * _G, _G)],
                                dst_v[r])

        @pl.when(g < 2)
        def _():
            for r in range(2):
                @pl.when(g == r)
                def _():
                    pltpu.sync_copy(dst_hbm.at[pl.ds(w * NCH + g * _G, _G)],
                                    dst_v[r])

        for r in range(2):
            @pl.when(g % 2 == r)
            def _():
                for j in range(_G):
                    pltpu.async_copy(buf, acc.at[dst_v[r].at[j]], sem,
                                     add=True)
        return 0

    lax.fori_loop(0, _NGRP, group, 0)
    for _k in range(2 * _G):
        pltpu.make_async_copy(ones_hbm, buf, sem).wait()
    plsc.subcore_barrier()
    pltpu.sync_copy(acc.at[pl.ds(base, ROWS_PER_TILE_ACC)],
                    out_hbm.at[c, pl.ds(base, ROWS_PER_TILE_ACC)])


_BLK = 400
_GRID = N // _BLK


def _tc1_body(deg_ref, x_ref, w_ref, dis_ref, hp_ref):
    deg = deg_ref[0] + deg_ref[1] + 1.0
    dis = lax.rsqrt(deg)
    dis_ref[...] = dis
    h = jnp.dot(x_ref[...], w_ref[...], preferred_element_type=jnp.float32)
    hp_ref[...] = h * dis[:, 0:1]


def _tc_mid_body(acc_ref, hp_ref, dis_ref, b_ref, w_ref, out_ref):
    dis = dis_ref[...][:, 0:1]
    t = (acc_ref[0] + acc_ref[1] + hp_ref[...]) * dis + b_ref[...]
    t = jnp.maximum(t, 0.0)
    out_ref[...] = jnp.dot(t, w_ref[...], preferred_element_type=jnp.float32) * dis


def _tc_final_body(acc_ref, hp_ref, dis_ref, b_ref, out_ref):
    dis = dis_ref[...][:, 0:1]
    out_ref[...] = (acc_ref[0] + acc_ref[1] + hp_ref[...]) * dis + b_ref[...]


def _tc1(deg, x, w):
    return pl.pallas_call(
        _tc1_body,
        grid=(_GRID,),
        in_specs=[
            pl.BlockSpec((NC, _BLK, 16), lambda i: (0, i, 0)),
            pl.BlockSpec((_BLK, D_IN), lambda i: (i, 0)),
            pl.BlockSpec((D_IN, D_HID), lambda i: (0, 0)),
        ],
        out_specs=[
            pl.BlockSpec((_BLK, 16), lambda i: (i, 0)),
            pl.BlockSpec((_BLK, D_HID), lambda i: (i, 0)),
        ],
        out_shape=[
            jax.ShapeDtypeStruct((N, 16), jnp.float32),
            jax.ShapeDtypeStruct((N, D_HID), jnp.float32),
        ],
    )(deg, x, w)


def _tc_mid(acc, hp, dis, b, w):
    d_in, d_out = w.shape
    return pl.pallas_call(
        _tc_mid_body,
        grid=(_GRID,),
        in_specs=[
            pl.BlockSpec((NC, _BLK, d_in), lambda i: (0, i, 0)),
            pl.BlockSpec((_BLK, d_in), lambda i: (i, 0)),
            pl.BlockSpec((_BLK, 16), lambda i: (i, 0)),
            pl.BlockSpec((1, d_in), lambda i: (0, 0)),
            pl.BlockSpec((d_in, d_out), lambda i: (0, 0)),
        ],
        out_specs=pl.BlockSpec((_BLK, d_out), lambda i: (i, 0)),
        out_shape=jax.ShapeDtypeStruct((N, d_out), jnp.float32),
    )(acc, hp, dis, b, w)


def _tc_final(acc, hp, dis, b):
    d = hp.shape[1]
    return pl.pallas_call(
        _tc_final_body,
        grid=(_GRID,),
        in_specs=[
            pl.BlockSpec((NC, _BLK, d), lambda i: (0, i, 0)),
            pl.BlockSpec((_BLK, d), lambda i: (i, 0)),
            pl.BlockSpec((_BLK, 16), lambda i: (i, 0)),
            pl.BlockSpec((1, d), lambda i: (0, 0)),
        ],
        out_specs=pl.BlockSpec((_BLK, d), lambda i: (i, 0)),
        out_shape=jax.ShapeDtypeStruct((N, d), jnp.float32),
    )(acc, hp, dis, b)


def kernel(x, edge_index, W1, b1, W2, b2, W_out, b_out):
    pad = E_PAD - E
    pad_i = jnp.arange(pad, dtype=jnp.int32)
    src = jnp.concatenate([edge_index[0], pad_i % N])
    dst = jnp.concatenate([edge_index[1], N + pad_i % (N_ACC - N)])
    src = src.reshape(NW * NCH, CH)
    dst = dst.reshape(NW * NCH, CH)
    zeros128 = jnp.zeros((ROWS_PER_TILE_ACC, D_HID), jnp.float32)
    ones_deg = jnp.ones((CH, _DEGW), jnp.float32)

    deg = _deg_kernel(dst, ones_deg, zeros128[:, :_DEGW])[:, :N, :16]
    dis, h1p = _tc1(deg, x, W1)
    acc1 = _edge_kernel_128(h1p, src, dst, zeros128)[:, :N]
    h2p = _tc_mid(acc1, h1p, dis, b1.reshape(1, D_HID), W2)
    acc2 = _edge_kernel_128(h2p, src, dst, zeros128)[:, :N]
    h3p = _tc_mid(acc2, h2p, dis, b2.reshape(1, D_HID), W_out)
    acc3 = _edge_kernel_64(h3p, src, dst, zeros128[:, :D_OUT])[:, :N]
    out = _tc_final(acc3, h3p, dis, b_out.reshape(1, D_OUT))
    return out
